# Initial kernel scaffold; baseline (speedup 1.0000x reference)
import functools

import jax
import jax.numpy as jnp
from jax import lax
from jax.experimental import pallas as pl
from jax.experimental.pallas import tpu as pltpu
from jax.experimental.pallas import tpu_sc as plsc

NC = 2
NS = 16
L = 16
NW = NC * NS


def _pad_up(v, m):
    return (v + m - 1) // m * m


def _make_deg_kernel(E_pad, NPAD):
    EPT = E_pad // NS
    CH = 2048
    NTCH = EPT // CH
    SL = NPAD // NS
    OUT_SL = NPAD // NW
    mesh = plsc.VectorSubcoreMesh(core_axis_name="c", subcore_axis_name="s")

    @functools.partial(
        pl.kernel,
        out_type=jax.ShapeDtypeStruct((NPAD,), jnp.float32),
        mesh=mesh,
        scratch_types=[
            pltpu.VMEM_SHARED((NPAD,), jnp.float32),
            pltpu.VMEM((CH,), jnp.int32),
            pltpu.VMEM((CH,), jnp.float32),
            pltpu.VMEM((SL,), jnp.float32),
        ],
    )
    def deg_kernel(dst_hbm, w_hbm, deg_out, deg_sh, didx, wv, initb):
        cid = lax.axis_index("c")
        sid = lax.axis_index("s")
        one = jnp.ones((L,), jnp.float32)

        def initbody(i, _):
            initb[pl.ds(i * L, L)] = one
            return 0

        lax.fori_loop(0, SL // L, initbody, 0)
        pltpu.sync_copy(initb, deg_sh.at[pl.ds(sid * SL, SL)])
        plsc.subcore_barrier()

        base = sid * EPT

        def chunk(ci, _):
            st = pl.multiple_of(base + ci * CH, 8)
            pltpu.sync_copy(dst_hbm.at[pl.ds(st, CH)], didx)
            pltpu.sync_copy(w_hbm.at[pl.ds(st, CH)], wv)
            pltpu.sync_copy(wv, deg_sh.at[didx], add=True)
            return 0

        lax.fori_loop(0, NTCH, chunk, 0)
        plsc.subcore_barrier()
        off = cid * (NPAD // NC) + sid * OUT_SL
        pltpu.sync_copy(deg_sh.at[pl.ds(off, OUT_SL)],
                        deg_out.at[pl.ds(off, OUT_SL)])

    return deg_kernel


def _make_agg_kernel(N, D, E_pad, NPAD):
    EPT = E_pad // NW
    B = 128
    NCH = EPT // B
    SL = NPAD // NS
    ZR = 64
    mesh = plsc.VectorSubcoreMesh(core_axis_name="c", subcore_axis_name="s")

    @functools.partial(
        pl.kernel,
        out_type=jax.ShapeDtypeStruct((NC, NPAD, D), jnp.float32),
        mesh=mesh,
        scratch_types=[
            pltpu.VMEM_SHARED((NPAD, D), jnp.float32),
            pltpu.VMEM((B,), jnp.int32),
            pltpu.VMEM((B,), jnp.int32),
            pltpu.VMEM((B,), jnp.float32),
            pltpu.VMEM((B, D), jnp.float32),
            pltpu.VMEM((ZR, D), jnp.float32),
            pltpu.SemaphoreType.DMA,
        ],
    )
    def agg_kernel(p_hbm, src_hbm, dst_hbm, w_hbm, q_out,
                   acc_sh, sidx, didx, wv, rows, zbuf, sem):
        cid = lax.axis_index("c")
        sid = lax.axis_index("s")
        zero = jnp.zeros((L,), jnp.float32)

        def zrow(r, _):
            for j in range(D // L):
                zbuf[r, pl.ds(j * L, L)] = zero
            return 0

        lax.fori_loop(0, ZR, zrow, 0)
        for k in range(SL // ZR):
            pltpu.sync_copy(zbuf, acc_sh.at[pl.ds(sid * SL + k * ZR, ZR)])
        plsc.subcore_barrier()

        base = (cid * NS + sid) * EPT

        def chunk(ci, _):
            st = pl.multiple_of(base + ci * B, 8)
            pltpu.sync_copy(src_hbm.at[pl.ds(st, B)], sidx)
            pltpu.sync_copy(dst_hbm.at[pl.ds(st, B)], didx)
            pltpu.sync_copy(w_hbm.at[pl.ds(st, B)], wv)
            pltpu.async_copy(p_hbm.at[sidx], rows, sem).wait()

            def sbody(e, _):
                ws = wv[e]
                for j in range(D // L):
                    sl = pl.ds(j * L, L)
                    rows[e, sl] = rows[e, sl] * ws
                return 0

            lax.fori_loop(0, B, sbody, 0)
            pltpu.sync_copy(rows, acc_sh.at[didx], add=True)
            return 0

        lax.fori_loop(0, NCH, chunk, 0)
        plsc.subcore_barrier()
        pltpu.sync_copy(acc_sh.at[pl.ds(sid * SL, SL)],
                        q_out.at[cid, pl.ds(sid * SL, SL)])

    return agg_kernel


def _tc1(x, W0, degr):
    N, D = x.shape
    BR = 1000
    grid = (N // BR,)

    def body(x_ref, w_ref, deg_ref, p_ref, dis_ref):
        dis = lax.rsqrt(deg_ref[...])
        h = jnp.dot(x_ref[...], w_ref[...], preferred_element_type=jnp.float32)
        p_ref[...] = h * dis
        dis_ref[...] = dis

    return pl.pallas_call(
        body,
        grid=grid,
        in_specs=[
            pl.BlockSpec((BR, D), lambda i: (i, 0)),
            pl.BlockSpec((D, D), lambda i: (0, 0)),
            pl.BlockSpec((BR, 1), lambda i: (i, 0)),
        ],
        out_specs=[
            pl.BlockSpec((BR, D), lambda i: (i, 0)),
            pl.BlockSpec((BR, 1), lambda i: (i, 0)),
        ],
        out_shape=[
            jax.ShapeDtypeStruct((N, D), jnp.float32),
            jax.ShapeDtypeStruct((N, 1), jnp.float32),
        ],
    )(x, W0, degr)


def _tc2(q1a, q1b, p0, dis, b0, W1):
    N, D = p0.shape
    BR = 1000
    grid = (N // BR,)

    def body(qa_ref, qb_ref, p_ref, dis_ref, b_ref, w_ref, p1_ref):
        dis = dis_ref[...]
        z = dis * (qa_ref[...] + qb_ref[...] + p_ref[...]) + b_ref[...]
        z = jnp.maximum(z, 0.0)
        p1_ref[...] = dis * jnp.dot(z, w_ref[...],
                                    preferred_element_type=jnp.float32)

    return pl.pallas_call(
        body,
        grid=grid,
        in_specs=[
            pl.BlockSpec((BR, D), lambda i: (i, 0)),
            pl.BlockSpec((BR, D), lambda i: (i, 0)),
            pl.BlockSpec((BR, D), lambda i: (i, 0)),
            pl.BlockSpec((BR, 1), lambda i: (i, 0)),
            pl.BlockSpec((1, D), lambda i: (0, 0)),
            pl.BlockSpec((D, D), lambda i: (0, 0)),
        ],
        out_specs=pl.BlockSpec((BR, D), lambda i: (i, 0)),
        out_shape=jax.ShapeDtypeStruct((N, D), jnp.float32),
    )(q1a, q1b, p0, dis, b0, W1)


def _tc3(q2a, q2b, p1, dis, b1):
    N, D = p1.shape
    BR = 1000
    grid = (N // BR,)

    def body(qa_ref, qb_ref, p_ref, dis_ref, b_ref, o_ref):
        o_ref[...] = (dis_ref[...] * (qa_ref[...] + qb_ref[...] + p_ref[...])
                      + b_ref[...])

    return pl.pallas_call(
        body,
        grid=grid,
        in_specs=[
            pl.BlockSpec((BR, D), lambda i: (i, 0)),
            pl.BlockSpec((BR, D), lambda i: (i, 0)),
            pl.BlockSpec((BR, D), lambda i: (i, 0)),
            pl.BlockSpec((BR, 1), lambda i: (i, 0)),
            pl.BlockSpec((1, D), lambda i: (0, 0)),
        ],
        out_specs=pl.BlockSpec((BR, D), lambda i: (i, 0)),
        out_shape=jax.ShapeDtypeStruct((N, D), jnp.float32),
    )(q2a, q2b, p1, dis, b1)


def kernel(x, edge_index, edge_weight, W0, b0, W1, b1):
    N, D = x.shape
    E = edge_index.shape[1]
    NPAD = _pad_up(N, 1024)
    E_pad = _pad_up(E, NS * 2048)
    pad = E_pad - E

    src = edge_index[0]
    dst = edge_index[1]
    srcp = jnp.concatenate([src, jnp.zeros((pad,), src.dtype)])
    dstp = jnp.concatenate([dst, jnp.full((pad,), NPAD - 1, dst.dtype)])
    wp = jnp.concatenate([edge_weight, jnp.zeros((pad,), edge_weight.dtype)])

    deg = _make_deg_kernel(E_pad, NPAD)(dstp, wp)
    degr = deg[:N].reshape(N, 1)

    p0, dis = _tc1(x, W0, degr)

    agg = _make_agg_kernel(N, D, E_pad, NPAD)
    q1 = agg(p0, srcp, dstp, wp)
    p1 = _tc2(q1[0, :N], q1[1, :N], p0, dis, b0.reshape(1, D), W1)
    q2 = agg(p1, srcp, dstp, wp)
    out = _tc3(q2[0, :N], q2[1, :N], p1, dis, b1.reshape(1, D))
    return out


# speedup vs baseline: 5.9180x; 5.9180x over previous
import functools

import jax
import jax.numpy as jnp
from jax import lax
from jax.experimental import pallas as pl
from jax.experimental.pallas import tpu as pltpu
from jax.experimental.pallas import tpu_sc as plsc

NC = 2
NS = 16
L = 16
NW = NC * NS


def _pad_up(v, m):
    return (v + m - 1) // m * m


def _make_deg_kernel(E_pad, NPAD):
    EPT = E_pad // NS
    CH = 2048
    NTCH = EPT // CH
    SL = NPAD // NS
    OUT_SL = NPAD // NW
    mesh = plsc.VectorSubcoreMesh(core_axis_name="c", subcore_axis_name="s")

    @functools.partial(
        pl.kernel,
        out_type=jax.ShapeDtypeStruct((NPAD,), jnp.float32),
        mesh=mesh,
        scratch_types=[
            pltpu.VMEM_SHARED((NPAD,), jnp.float32),
            pltpu.VMEM((CH,), jnp.int32),
            pltpu.VMEM((CH,), jnp.float32),
            pltpu.VMEM((SL,), jnp.float32),
        ],
    )
    def deg_kernel(dst_hbm, w_hbm, deg_out, deg_sh, didx, wv, initb):
        cid = lax.axis_index("c")
        sid = lax.axis_index("s")
        one = jnp.ones((L,), jnp.float32)

        def initbody(i, _):
            initb[pl.ds(i * L, L)] = one
            return 0

        lax.fori_loop(0, SL // L, initbody, 0)
        pltpu.sync_copy(initb, deg_sh.at[pl.ds(sid * SL, SL)])
        plsc.subcore_barrier()

        base = sid * EPT

        def chunk(ci, _):
            st = pl.multiple_of(base + ci * CH, 8)
            pltpu.sync_copy(dst_hbm.at[pl.ds(st, CH)], didx)
            pltpu.sync_copy(w_hbm.at[pl.ds(st, CH)], wv)
            pltpu.sync_copy(wv, deg_sh.at[didx], add=True)
            return 0

        lax.fori_loop(0, NTCH, chunk, 0)
        plsc.subcore_barrier()
        off = cid * (NPAD // NC) + sid * OUT_SL
        pltpu.sync_copy(deg_sh.at[pl.ds(off, OUT_SL)],
                        initb.at[pl.ds(0, OUT_SL)])
        pltpu.sync_copy(initb.at[pl.ds(0, OUT_SL)],
                        deg_out.at[pl.ds(off, OUT_SL)])

    return deg_kernel


def _make_agg_kernel(N, D, E_pad, NPAD):
    EPT = E_pad // NW
    B = 128
    NCH = EPT // B
    SL = NPAD // NS
    ZR = 64
    mesh = plsc.VectorSubcoreMesh(core_axis_name="c", subcore_axis_name="s")

    @functools.partial(
        pl.kernel,
        out_type=jax.ShapeDtypeStruct((NC, NPAD, D), jnp.float32),
        mesh=mesh,
        scratch_types=[
            pltpu.VMEM_SHARED((NPAD, D), jnp.float32),
            pltpu.VMEM((B,), jnp.int32),
            pltpu.VMEM((B,), jnp.int32),
            pltpu.VMEM((B,), jnp.float32),
            pltpu.VMEM((B, D), jnp.float32),
            pltpu.VMEM((ZR, D), jnp.float32),
            pltpu.SemaphoreType.DMA,
        ],
    )
    def agg_kernel(p_hbm, src_hbm, dst_hbm, w_hbm, q_out,
                   acc_sh, sidx, didx, wv, rows, zbuf, sem):
        cid = lax.axis_index("c")
        sid = lax.axis_index("s")
        zero = jnp.zeros((L,), jnp.float32)

        def zrow(r, _):
            for j in range(D // L):
                zbuf[r, pl.ds(j * L, L)] = zero
            return 0

        lax.fori_loop(0, ZR, zrow, 0)
        for k in range(SL // ZR):
            pltpu.sync_copy(zbuf, acc_sh.at[pl.ds(sid * SL + k * ZR, ZR)])
        plsc.subcore_barrier()

        base = (cid * NS + sid) * EPT

        def chunk(ci, _):
            st = pl.multiple_of(base + ci * B, 8)
            pltpu.sync_copy(src_hbm.at[pl.ds(st, B)], sidx)
            pltpu.sync_copy(dst_hbm.at[pl.ds(st, B)], didx)
            pltpu.sync_copy(w_hbm.at[pl.ds(st, B)], wv)
            pltpu.async_copy(p_hbm.at[sidx], rows, sem).wait()

            def sbody(g, _):
                wg = wv[pl.ds(g * L, L)]
                for lane in range(L):
                    ws = wg[lane]
                    e = g * L + lane
                    for j in range(D // L):
                        sl = pl.ds(j * L, L)
                        rows[e, sl] = rows[e, sl] * ws
                return 0

            lax.fori_loop(0, B // L, sbody, 0)
            pltpu.sync_copy(rows, acc_sh.at[didx], add=True)
            return 0

        lax.fori_loop(0, NCH, chunk, 0)
        plsc.subcore_barrier()
        for k in range(SL // ZR):
            r0 = sid * SL + k * ZR
            pltpu.sync_copy(acc_sh.at[pl.ds(r0, ZR)], zbuf)
            pltpu.sync_copy(zbuf, q_out.at[cid, pl.ds(r0, ZR)])

    return agg_kernel


def _tc1(x, W0, degr):
    N, D = x.shape
    BR = 1000
    grid = (N // BR,)

    def body(x_ref, w_ref, deg_ref, p_ref, dis_ref):
        dis = lax.rsqrt(deg_ref[...])
        h = jnp.dot(x_ref[...], w_ref[...], preferred_element_type=jnp.float32)
        p_ref[...] = h * dis
        dis_ref[...] = dis

    return pl.pallas_call(
        body,
        grid=grid,
        in_specs=[
            pl.BlockSpec((BR, D), lambda i: (i, 0)),
            pl.BlockSpec((D, D), lambda i: (0, 0)),
            pl.BlockSpec((BR, 1), lambda i: (i, 0)),
        ],
        out_specs=[
            pl.BlockSpec((BR, D), lambda i: (i, 0)),
            pl.BlockSpec((BR, 1), lambda i: (i, 0)),
        ],
        out_shape=[
            jax.ShapeDtypeStruct((N, D), jnp.float32),
            jax.ShapeDtypeStruct((N, 1), jnp.float32),
        ],
    )(x, W0, degr)


def _tc2(q1a, q1b, p0, dis, b0, W1):
    N, D = p0.shape
    BR = 1000
    grid = (N // BR,)

    def body(qa_ref, qb_ref, p_ref, dis_ref, b_ref, w_ref, p1_ref):
        dis = dis_ref[...]
        z = dis * (qa_ref[...] + qb_ref[...] + p_ref[...]) + b_ref[...]
        z = jnp.maximum(z, 0.0)
        p1_ref[...] = dis * jnp.dot(z, w_ref[...],
                                    preferred_element_type=jnp.float32)

    return pl.pallas_call(
        body,
        grid=grid,
        in_specs=[
            pl.BlockSpec((BR, D), lambda i: (i, 0)),
            pl.BlockSpec((BR, D), lambda i: (i, 0)),
            pl.BlockSpec((BR, D), lambda i: (i, 0)),
            pl.BlockSpec((BR, 1), lambda i: (i, 0)),
            pl.BlockSpec((1, D), lambda i: (0, 0)),
            pl.BlockSpec((D, D), lambda i: (0, 0)),
        ],
        out_specs=pl.BlockSpec((BR, D), lambda i: (i, 0)),
        out_shape=jax.ShapeDtypeStruct((N, D), jnp.float32),
    )(q1a, q1b, p0, dis, b0, W1)


def _tc3(q2a, q2b, p1, dis, b1):
    N, D = p1.shape
    BR = 1000
    grid = (N // BR,)

    def body(qa_ref, qb_ref, p_ref, dis_ref, b_ref, o_ref):
        o_ref[...] = (dis_ref[...] * (qa_ref[...] + qb_ref[...] + p_ref[...])
                      + b_ref[...])

    return pl.pallas_call(
        body,
        grid=grid,
        in_specs=[
            pl.BlockSpec((BR, D), lambda i: (i, 0)),
            pl.BlockSpec((BR, D), lambda i: (i, 0)),
            pl.BlockSpec((BR, D), lambda i: (i, 0)),
            pl.BlockSpec((BR, 1), lambda i: (i, 0)),
            pl.BlockSpec((1, D), lambda i: (0, 0)),
        ],
        out_specs=pl.BlockSpec((BR, D), lambda i: (i, 0)),
        out_shape=jax.ShapeDtypeStruct((N, D), jnp.float32),
    )(q2a, q2b, p1, dis, b1)


def kernel(x, edge_index, edge_weight, W0, b0, W1, b1):
    N, D = x.shape
    E = edge_index.shape[1]
    NPAD = _pad_up(N, 1024)
    E_pad = _pad_up(E, NS * 2048)
    pad = E_pad - E

    src = edge_index[0]
    dst = edge_index[1]
    srcp = jnp.concatenate([src, jnp.zeros((pad,), src.dtype)])
    dstp = jnp.concatenate([dst, jnp.full((pad,), NPAD - 1, dst.dtype)])
    wp = jnp.concatenate([edge_weight, jnp.zeros((pad,), edge_weight.dtype)])

    deg = _make_deg_kernel(E_pad, NPAD)(dstp, wp)
    degr = deg[:N].reshape(N, 1)

    p0, dis = _tc1(x, W0, degr)

    agg = _make_agg_kernel(N, D, E_pad, NPAD)
    q1 = agg(p0, srcp, dstp, wp)
    p1 = _tc2(q1[0, :N], q1[1, :N], p0, dis, b0.reshape(1, D), W1)
    q2 = agg(p1, srcp, dstp, wp)
    out = _tc3(q2[0, :N], q2[1, :N], p1, dis, b1.reshape(1, D))
    return out


# speedup vs baseline: 9.6938x; 1.6380x over previous
import functools

import jax
import jax.numpy as jnp
from jax import lax
from jax.experimental import pallas as pl
from jax.experimental.pallas import tpu as pltpu
from jax.experimental.pallas import tpu_sc as plsc

NC = 2
NS = 16
L = 16
NW = NC * NS


def _pad_up(v, m):
    return (v + m - 1) // m * m


def _make_deg_kernel(E_pad, NPAD):
    EPT = E_pad // NS
    CH = 2048
    NTCH = EPT // CH
    SL = NPAD // NS
    OUT_SL = NPAD // NW
    mesh = plsc.VectorSubcoreMesh(core_axis_name="c", subcore_axis_name="s")

    @functools.partial(
        pl.kernel,
        out_type=jax.ShapeDtypeStruct((NPAD,), jnp.float32),
        mesh=mesh,
        scratch_types=[
            pltpu.VMEM_SHARED((NPAD,), jnp.float32),
            pltpu.VMEM((CH,), jnp.int32),
            pltpu.VMEM((CH,), jnp.float32),
            pltpu.VMEM((SL,), jnp.float32),
        ],
    )
    def deg_kernel(dst_hbm, w_hbm, deg_out, deg_sh, didx, wv, initb):
        cid = lax.axis_index("c")
        sid = lax.axis_index("s")
        one = jnp.ones((L,), jnp.float32)

        def initbody(i, _):
            initb[pl.ds(i * L, L)] = one
            return 0

        lax.fori_loop(0, SL // L, initbody, 0)
        pltpu.sync_copy(initb, deg_sh.at[pl.ds(sid * SL, SL)])
        plsc.subcore_barrier()

        base = sid * EPT

        def chunk(ci, _):
            st = pl.multiple_of(base + ci * CH, 8)
            pltpu.sync_copy(dst_hbm.at[pl.ds(st, CH)], didx)
            pltpu.sync_copy(w_hbm.at[pl.ds(st, CH)], wv)
            pltpu.sync_copy(wv, deg_sh.at[didx], add=True)
            return 0

        lax.fori_loop(0, NTCH, chunk, 0)
        plsc.subcore_barrier()
        off = cid * (NPAD // NC) + sid * OUT_SL
        pltpu.sync_copy(deg_sh.at[pl.ds(off, OUT_SL)],
                        initb.at[pl.ds(0, OUT_SL)])
        pltpu.sync_copy(initb.at[pl.ds(0, OUT_SL)],
                        deg_out.at[pl.ds(off, OUT_SL)])

    return deg_kernel


def _make_agg_kernel(N, D, E_pad, NPAD):
    EPT = E_pad // NW
    B = 128
    NCH = EPT // B
    SUP = 16
    NSUP = NCH // SUP
    SL = NPAD // NS
    mesh = plsc.VectorSubcoreMesh(core_axis_name="c", subcore_axis_name="s")

    @functools.partial(
        pl.kernel,
        out_type=jax.ShapeDtypeStruct((NC, NPAD, D), jnp.float32),
        mesh=mesh,
        scratch_types=[
            pltpu.VMEM_SHARED((NPAD, D), jnp.float32),
            pltpu.VMEM((2, SUP, B), jnp.int32),
            pltpu.VMEM((2, SUP, B), jnp.int32),
            pltpu.VMEM((2, SUP, B), jnp.float32),
            pltpu.VMEM((B, D), jnp.float32),
            pltpu.VMEM((B, D), jnp.float32),
            pltpu.SemaphoreType.DMA,
            pltpu.SemaphoreType.DMA,
            pltpu.SemaphoreType.DMA,
            pltpu.SemaphoreType.DMA,
            pltpu.SemaphoreType.DMA,
            pltpu.SemaphoreType.DMA,
        ],
    )
    def agg_kernel(p_hbm, src_hbm, dst_hbm, w_hbm, q_out,
                   acc_sh, sidx, didx, wv, rows0, rows1,
                   g0, g1, s0, s1, ms0, ms1):
        cid = lax.axis_index("c")
        sid = lax.axis_index("s")
        trow = (cid * NS + sid) * NCH
        msems = (ms0, ms1)

        def stage_meta(sup, par):
            r = pl.multiple_of(trow + sup * SUP, 8)
            sem = msems[par]
            return (
                pltpu.async_copy(src_hbm.at[pl.ds(r, SUP)], sidx.at[par], sem),
                pltpu.async_copy(dst_hbm.at[pl.ds(r, SUP)], didx.at[par], sem),
                pltpu.async_copy(w_hbm.at[pl.ds(r, SUP)], wv.at[par], sem),
            )

        cps = stage_meta(0, 0)

        zero = jnp.zeros((L,), jnp.float32)

        def zrow(r, _):
            for j in range(D // L):
                rows0[r, pl.ds(j * L, L)] = zero
            return 0

        lax.fori_loop(0, B, zrow, 0)
        for k in range(SL // B):
            pltpu.sync_copy(rows0, acc_sh.at[pl.ds(sid * SL + k * B, B)])
        plsc.subcore_barrier()

        def scale(rbuf, wa, c):
            def sbody(g, _):
                wg = wa[c, pl.ds(g * L, L)]
                for lane in range(L):
                    ws = wg[lane]
                    e = g * L + lane
                    for j in range(D // L):
                        sl = pl.ds(j * L, L)
                        rbuf[e, sl] = rbuf[e, sl] * ws
                return 0

            lax.fori_loop(0, B // L, sbody, 0)

        for sup in range(NSUP):
            par = sup % 2
            sA = sidx.at[par]
            dA = didx.at[par]
            wA = wv.at[par]
            if sup + 1 < NSUP:
                next_cps = stage_meta(sup + 1, 1 - par)
            for cp in cps:
                cp.wait()
            cps = next_cps if sup + 1 < NSUP else ()

            pltpu.async_copy(p_hbm.at[sA.at[0]], rows0, g0)

            def chunk2(cc, _):
                c0 = cc * 2
                c1 = c0 + 1
                pltpu.make_async_copy(p_hbm.at[sA.at[c0]], rows0, g0).wait()

                @pl.when(cc >= 1)
                def _():
                    pltpu.make_async_copy(rows1, acc_sh.at[dA.at[c0 - 1]],
                                          s1).wait()

                pltpu.async_copy(p_hbm.at[sA.at[c1]], rows1, g1)
                scale(rows0, wA, c0)
                pltpu.async_copy(rows0, acc_sh.at[dA.at[c0]], s0, add=True)

                pltpu.make_async_copy(p_hbm.at[sA.at[c1]], rows1, g1).wait()

                @pl.when(cc < SUP // 2 - 1)
                def _():
                    pltpu.make_async_copy(rows0, acc_sh.at[dA.at[c0]],
                                          s0).wait()
                    pltpu.async_copy(p_hbm.at[sA.at[c0 + 2]], rows0, g0)

                scale(rows1, wA, c1)
                pltpu.async_copy(rows1, acc_sh.at[dA.at[c1]], s1, add=True)
                return 0

            lax.fori_loop(0, SUP // 2, chunk2, 0)
            pltpu.make_async_copy(rows0, acc_sh.at[dA.at[SUP - 2]],
                                  s0).wait()
            pltpu.make_async_copy(rows1, acc_sh.at[dA.at[SUP - 1]],
                                  s1).wait()

        plsc.subcore_barrier()
        for k in range(SL // B):
            r0 = sid * SL + k * B
            pltpu.sync_copy(acc_sh.at[pl.ds(r0, B)], rows0)
            pltpu.sync_copy(rows0, q_out.at[cid, pl.ds(r0, B)])

    return agg_kernel


def _tc1(x, W0, degr):
    N, D = x.shape
    BR = 1000
    grid = (N // BR,)

    def body(x_ref, w_ref, deg_ref, p_ref, dis_ref):
        dis = lax.rsqrt(deg_ref[...])
        h = jnp.dot(x_ref[...], w_ref[...], preferred_element_type=jnp.float32)
        p_ref[...] = h * dis
        dis_ref[...] = dis

    return pl.pallas_call(
        body,
        grid=grid,
        in_specs=[
            pl.BlockSpec((BR, D), lambda i: (i, 0)),
            pl.BlockSpec((D, D), lambda i: (0, 0)),
            pl.BlockSpec((BR, 1), lambda i: (i, 0)),
        ],
        out_specs=[
            pl.BlockSpec((BR, D), lambda i: (i, 0)),
            pl.BlockSpec((BR, 1), lambda i: (i, 0)),
        ],
        out_shape=[
            jax.ShapeDtypeStruct((N, D), jnp.float32),
            jax.ShapeDtypeStruct((N, 1), jnp.float32),
        ],
    )(x, W0, degr)


def _tc2(q1a, q1b, p0, dis, b0, W1):
    N, D = p0.shape
    BR = 1000
    grid = (N // BR,)

    def body(qa_ref, qb_ref, p_ref, dis_ref, b_ref, w_ref, p1_ref):
        dis = dis_ref[...]
        z = dis * (qa_ref[...] + qb_ref[...] + p_ref[...]) + b_ref[...]
        z = jnp.maximum(z, 0.0)
        p1_ref[...] = dis * jnp.dot(z, w_ref[...],
                                    preferred_element_type=jnp.float32)

    return pl.pallas_call(
        body,
        grid=grid,
        in_specs=[
            pl.BlockSpec((BR, D), lambda i: (i, 0)),
            pl.BlockSpec((BR, D), lambda i: (i, 0)),
            pl.BlockSpec((BR, D), lambda i: (i, 0)),
            pl.BlockSpec((BR, 1), lambda i: (i, 0)),
            pl.BlockSpec((1, D), lambda i: (0, 0)),
            pl.BlockSpec((D, D), lambda i: (0, 0)),
        ],
        out_specs=pl.BlockSpec((BR, D), lambda i: (i, 0)),
        out_shape=jax.ShapeDtypeStruct((N, D), jnp.float32),
    )(q1a, q1b, p0, dis, b0, W1)


def _tc3(q2a, q2b, p1, dis, b1):
    N, D = p1.shape
    BR = 1000
    grid = (N // BR,)

    def body(qa_ref, qb_ref, p_ref, dis_ref, b_ref, o_ref):
        o_ref[...] = (dis_ref[...] * (qa_ref[...] + qb_ref[...] + p_ref[...])
                      + b_ref[...])

    return pl.pallas_call(
        body,
        grid=grid,
        in_specs=[
            pl.BlockSpec((BR, D), lambda i: (i, 0)),
            pl.BlockSpec((BR, D), lambda i: (i, 0)),
            pl.BlockSpec((BR, D), lambda i: (i, 0)),
            pl.BlockSpec((BR, 1), lambda i: (i, 0)),
            pl.BlockSpec((1, D), lambda i: (0, 0)),
        ],
        out_specs=pl.BlockSpec((BR, D), lambda i: (i, 0)),
        out_shape=jax.ShapeDtypeStruct((N, D), jnp.float32),
    )(q2a, q2b, p1, dis, b1)


def kernel(x, edge_index, edge_weight, W0, b0, W1, b1):
    N, D = x.shape
    E = edge_index.shape[1]
    NPAD = _pad_up(N, 1024)
    E_pad = _pad_up(E, NS * 2048)
    pad = E_pad - E

    src = edge_index[0]
    dst = edge_index[1]
    srcp = jnp.concatenate([src, jnp.zeros((pad,), src.dtype)])
    dstp = jnp.concatenate([dst, jnp.full((pad,), NPAD - 1, dst.dtype)])
    wp = jnp.concatenate([edge_weight, jnp.zeros((pad,), edge_weight.dtype)])

    deg = _make_deg_kernel(E_pad, NPAD)(dstp, wp)
    degr = deg[:N].reshape(N, 1)

    p0, dis = _tc1(x, W0, degr)

    agg = _make_agg_kernel(N, D, E_pad, NPAD)
    src2 = srcp.reshape(-1, 128)
    dst2 = dstp.reshape(-1, 128)
    w2 = wp.reshape(-1, 128)
    q1 = agg(p0, src2, dst2, w2)
    p1 = _tc2(q1[0, :N], q1[1, :N], p0, dis, b0.reshape(1, D), W1)
    q2 = agg(p1, src2, dst2, w2)
    out = _tc3(q2[0, :N], q2[1, :N], p1, dis, b1.reshape(1, D))
    return out


# speedup vs baseline: 26.4853x; 2.7322x over previous
import functools

import jax
import jax.numpy as jnp
from jax import lax
from jax.experimental import pallas as pl
from jax.experimental.pallas import tpu as pltpu
from jax.experimental.pallas import tpu_sc as plsc

NC = 2
NS = 16
L = 16
NW = NC * NS


def _pad_up(v, m):
    return (v + m - 1) // m * m


def _make_deg_kernel(E_pad, NPAD):
    EPT = E_pad // NS
    CH = 2048
    NTCH = EPT // CH
    SL = NPAD // NS
    OUT_SL = NPAD // NW
    mesh = plsc.VectorSubcoreMesh(core_axis_name="c", subcore_axis_name="s")

    @functools.partial(
        pl.kernel,
        out_type=jax.ShapeDtypeStruct((NPAD,), jnp.float32),
        mesh=mesh,
        scratch_types=[
            pltpu.VMEM_SHARED((NPAD,), jnp.float32),
            pltpu.VMEM((CH,), jnp.int32),
            pltpu.VMEM((CH,), jnp.float32),
            pltpu.VMEM((SL,), jnp.float32),
        ],
    )
    def deg_kernel(dst_hbm, w_hbm, deg_out, deg_sh, didx, wv, initb):
        cid = lax.axis_index("c")
        sid = lax.axis_index("s")
        one = jnp.ones((L,), jnp.float32)

        def initbody(i, _):
            initb[pl.ds(i * L, L)] = one
            return 0

        lax.fori_loop(0, SL // L, initbody, 0)
        pltpu.sync_copy(initb, deg_sh.at[pl.ds(sid * SL, SL)])
        plsc.subcore_barrier()

        base = sid * EPT

        def chunk(ci, _):
            st = pl.multiple_of(base + ci * CH, 8)
            pltpu.sync_copy(dst_hbm.at[pl.ds(st, CH)], didx)
            pltpu.sync_copy(w_hbm.at[pl.ds(st, CH)], wv)
            pltpu.sync_copy(wv, deg_sh.at[didx], add=True)
            return 0

        lax.fori_loop(0, NTCH, chunk, 0)
        plsc.subcore_barrier()
        off = cid * (NPAD // NC) + sid * OUT_SL
        pltpu.sync_copy(deg_sh.at[pl.ds(off, OUT_SL)],
                        initb.at[pl.ds(0, OUT_SL)])
        pltpu.sync_copy(initb.at[pl.ds(0, OUT_SL)],
                        deg_out.at[pl.ds(off, OUT_SL)])

    return deg_kernel


def _make_agg_kernel(N, D, E_pad, NPAD):
    EPT = E_pad // NW
    B = 128
    NCH = EPT // B
    SUP = 16
    NSUP = NCH // SUP
    SL = NPAD // NS
    mesh = plsc.VectorSubcoreMesh(core_axis_name="c", subcore_axis_name="s")

    @functools.partial(
        pl.kernel,
        out_type=jax.ShapeDtypeStruct((NC, NPAD, D), jnp.float32),
        mesh=mesh,
        scratch_types=[
            pltpu.VMEM_SHARED((NPAD, D), jnp.float32),
            pltpu.VMEM((2, SUP, B), jnp.int32),
            pltpu.VMEM((2, SUP, B), jnp.int32),
            pltpu.VMEM((2, SUP, B), jnp.float32),
            pltpu.VMEM((B, D), jnp.float32),
            pltpu.VMEM((B, D), jnp.float32),
            pltpu.SemaphoreType.DMA,
            pltpu.SemaphoreType.DMA,
            pltpu.SemaphoreType.DMA,
            pltpu.SemaphoreType.DMA,
            pltpu.SemaphoreType.DMA,
            pltpu.SemaphoreType.DMA,
        ],
    )
    def agg_kernel(p_hbm, src_hbm, dst_hbm, w_hbm, q_out,
                   acc_sh, sidx, didx, wv, rows0, rows1,
                   g0, g1, s0, s1, ms0, ms1):
        cid = lax.axis_index("c")
        sid = lax.axis_index("s")
        trow = (cid * NS + sid) * NCH
        msems = (ms0, ms1)

        def stage_meta(sup, par):
            r = pl.multiple_of(trow + sup * SUP, 8)
            sem = msems[par]
            return (
                pltpu.async_copy(src_hbm.at[pl.ds(r, SUP)], sidx.at[par], sem),
                pltpu.async_copy(dst_hbm.at[pl.ds(r, SUP)], didx.at[par], sem),
                pltpu.async_copy(w_hbm.at[pl.ds(r, SUP)], wv.at[par], sem),
            )

        cps = stage_meta(0, 0)

        zero = jnp.zeros((L,), jnp.float32)

        def zrow(r, _):
            for j in range(D // L):
                rows0[r, pl.ds(j * L, L)] = zero
            return 0

        lax.fori_loop(0, B, zrow, 0)
        zcps = [
            pltpu.async_copy(rows0, acc_sh.at[pl.ds(sid * SL + k * B, B)],
                             s0)
            for k in range(SL // B)
        ]
        for cp in zcps:
            cp.wait()
        plsc.subcore_barrier()

        def scale(rbuf, wa, c):
            def sbody(g, _):
                wg = wa[c, pl.ds(g * L, L)]
                for lane in range(L):
                    ws = wg[lane]
                    e = g * L + lane
                    for j in range(D // L):
                        sl = pl.ds(j * L, L)
                        rbuf[e, sl] = rbuf[e, sl] * ws
                return 0

            lax.fori_loop(0, B // L, sbody, 0)

        for sup in range(NSUP):
            par = sup % 2
            sA = sidx.at[par]
            dA = didx.at[par]
            wA = wv.at[par]
            if sup + 1 < NSUP:
                next_cps = stage_meta(sup + 1, 1 - par)
            for cp in cps:
                cp.wait()
            cps = next_cps if sup + 1 < NSUP else ()

            pltpu.async_copy(p_hbm.at[sA.at[0]], rows0, g0)

            def chunk2(cc, _):
                c0 = cc * 2
                c1 = c0 + 1
                pltpu.make_async_copy(p_hbm.at[sA.at[c0]], rows0, g0).wait()

                @pl.when(cc >= 1)
                def _():
                    pltpu.make_async_copy(rows1, acc_sh.at[dA.at[c0 - 1]],
                                          s1).wait()

                pltpu.async_copy(p_hbm.at[sA.at[c1]], rows1, g1)
                scale(rows0, wA, c0)
                pltpu.async_copy(rows0, acc_sh.at[dA.at[c0]], s0, add=True)

                pltpu.make_async_copy(p_hbm.at[sA.at[c1]], rows1, g1).wait()

                @pl.when(cc < SUP // 2 - 1)
                def _():
                    pltpu.make_async_copy(rows0, acc_sh.at[dA.at[c0]],
                                          s0).wait()
                    pltpu.async_copy(p_hbm.at[sA.at[c0 + 2]], rows0, g0)

                scale(rows1, wA, c1)
                pltpu.async_copy(rows1, acc_sh.at[dA.at[c1]], s1, add=True)
                return 0

            lax.fori_loop(0, SUP // 2, chunk2, 0)
            pltpu.make_async_copy(rows0, acc_sh.at[dA.at[SUP - 2]],
                                  s0).wait()
            pltpu.make_async_copy(rows1, acc_sh.at[dA.at[SUP - 1]],
                                  s1).wait()

        plsc.subcore_barrier()
        wbufs = (rows0, rows1)
        wsems = (g0, g1)
        osems = (s0, s1)
        outs = [None, None]
        for k in range(SL // B):
            r0 = sid * SL + k * B
            buf = wbufs[k % 2]
            if outs[k % 2] is not None:
                outs[k % 2].wait()
            pltpu.async_copy(acc_sh.at[pl.ds(r0, B)], buf,
                             wsems[k % 2]).wait()
            outs[k % 2] = pltpu.async_copy(buf, q_out.at[cid, pl.ds(r0, B)],
                                           osems[k % 2])
        for cp in outs:
            cp.wait()

    return agg_kernel


def _tc1(x, W0, degr):
    N, D = x.shape
    BR = 1000
    grid = (N // BR,)

    def body(x_ref, w_ref, deg_ref, p_ref, dis_ref):
        dis = lax.rsqrt(deg_ref[...])
        h = jnp.dot(x_ref[...], w_ref[...], preferred_element_type=jnp.float32)
        p_ref[...] = h * dis
        dis_ref[...] = dis

    return pl.pallas_call(
        body,
        grid=grid,
        in_specs=[
            pl.BlockSpec((BR, D), lambda i: (i, 0)),
            pl.BlockSpec((D, D), lambda i: (0, 0)),
            pl.BlockSpec((BR, 1), lambda i: (i, 0)),
        ],
        out_specs=[
            pl.BlockSpec((BR, D), lambda i: (i, 0)),
            pl.BlockSpec((BR, 1), lambda i: (i, 0)),
        ],
        out_shape=[
            jax.ShapeDtypeStruct((N, D), jnp.float32),
            jax.ShapeDtypeStruct((N, 1), jnp.float32),
        ],
    )(x, W0, degr)


def _tc2(q1a, q1b, p0, dis, b0, W1):
    N, D = p0.shape
    BR = 1000
    grid = (N // BR,)

    def body(qa_ref, qb_ref, p_ref, dis_ref, b_ref, w_ref, p1_ref):
        dis = dis_ref[...]
        z = dis * (qa_ref[...] + qb_ref[...] + p_ref[...]) + b_ref[...]
        z = jnp.maximum(z, 0.0)
        p1_ref[...] = dis * jnp.dot(z, w_ref[...],
                                    preferred_element_type=jnp.float32)

    return pl.pallas_call(
        body,
        grid=grid,
        in_specs=[
            pl.BlockSpec((BR, D), lambda i: (i, 0)),
            pl.BlockSpec((BR, D), lambda i: (i, 0)),
            pl.BlockSpec((BR, D), lambda i: (i, 0)),
            pl.BlockSpec((BR, 1), lambda i: (i, 0)),
            pl.BlockSpec((1, D), lambda i: (0, 0)),
            pl.BlockSpec((D, D), lambda i: (0, 0)),
        ],
        out_specs=pl.BlockSpec((BR, D), lambda i: (i, 0)),
        out_shape=jax.ShapeDtypeStruct((N, D), jnp.float32),
    )(q1a, q1b, p0, dis, b0, W1)


def _tc3(q2a, q2b, p1, dis, b1):
    N, D = p1.shape
    BR = 1000
    grid = (N // BR,)

    def body(qa_ref, qb_ref, p_ref, dis_ref, b_ref, o_ref):
        o_ref[...] = (dis_ref[...] * (qa_ref[...] + qb_ref[...] + p_ref[...])
                      + b_ref[...])

    return pl.pallas_call(
        body,
        grid=grid,
        in_specs=[
            pl.BlockSpec((BR, D), lambda i: (i, 0)),
            pl.BlockSpec((BR, D), lambda i: (i, 0)),
            pl.BlockSpec((BR, D), lambda i: (i, 0)),
            pl.BlockSpec((BR, 1), lambda i: (i, 0)),
            pl.BlockSpec((1, D), lambda i: (0, 0)),
        ],
        out_specs=pl.BlockSpec((BR, D), lambda i: (i, 0)),
        out_shape=jax.ShapeDtypeStruct((N, D), jnp.float32),
    )(q2a, q2b, p1, dis, b1)


def kernel(x, edge_index, edge_weight, W0, b0, W1, b1):
    N, D = x.shape
    E = edge_index.shape[1]
    NPAD = _pad_up(N, 1024)
    E_pad = _pad_up(E, NS * 2048)
    pad = E_pad - E

    src = edge_index[0]
    dst = edge_index[1]
    apad = jnp.arange(pad, dtype=dst.dtype)
    srcp = jnp.concatenate([src, apad % N])
    dstp = jnp.concatenate([dst, N + apad % (NPAD - N)])
    wp = jnp.concatenate([edge_weight, jnp.zeros((pad,), edge_weight.dtype)])

    deg = _make_deg_kernel(E_pad, NPAD)(dstp, wp)
    degr = deg[:N].reshape(N, 1)

    p0, dis = _tc1(x, W0, degr)

    agg = _make_agg_kernel(N, D, E_pad, NPAD)
    src2 = srcp.reshape(-1, 128)
    dst2 = dstp.reshape(-1, 128)
    w2 = wp.reshape(-1, 128)
    q1 = agg(p0, src2, dst2, w2)
    p1 = _tc2(q1[0, :N], q1[1, :N], p0, dis, b0.reshape(1, D), W1)
    q2 = agg(p1, src2, dst2, w2)
    out = _tc3(q2[0, :N], q2[1, :N], p1, dis, b1.reshape(1, D))
    return out


# speedup vs baseline: 26.5940x; 1.0041x over previous
import functools

import jax
import jax.numpy as jnp
from jax import lax
from jax.experimental import pallas as pl
from jax.experimental.pallas import tpu as pltpu
from jax.experimental.pallas import tpu_sc as plsc

NC = 2
NS = 16
L = 16
NW = NC * NS


def _pad_up(v, m):
    return (v + m - 1) // m * m


def _make_deg_kernel(E_pad, NPAD):
    EPT = E_pad // NS
    CH = 2048
    NTCH = EPT // CH
    SL = NPAD // NS
    OUT_SL = NPAD // NW
    mesh = plsc.VectorSubcoreMesh(core_axis_name="c", subcore_axis_name="s")

    @functools.partial(
        pl.kernel,
        out_type=jax.ShapeDtypeStruct((NPAD,), jnp.float32),
        mesh=mesh,
        scratch_types=[
            pltpu.VMEM_SHARED((NPAD,), jnp.float32),
            pltpu.VMEM((CH,), jnp.int32),
            pltpu.VMEM((CH,), jnp.float32),
            pltpu.VMEM((SL,), jnp.float32),
        ],
    )
    def deg_kernel(dst_hbm, w_hbm, deg_out, deg_sh, didx, wv, initb):
        cid = lax.axis_index("c")
        sid = lax.axis_index("s")
        one = jnp.ones((L,), jnp.float32)

        def initbody(i, _):
            initb[pl.ds(i * L, L)] = one
            return 0

        lax.fori_loop(0, SL // L, initbody, 0)
        pltpu.sync_copy(initb, deg_sh.at[pl.ds(sid * SL, SL)])
        plsc.subcore_barrier()

        base = sid * EPT

        def chunk(ci, _):
            st = pl.multiple_of(base + ci * CH, 8)
            pltpu.sync_copy(dst_hbm.at[pl.ds(st, CH)], didx)
            pltpu.sync_copy(w_hbm.at[pl.ds(st, CH)], wv)
            pltpu.sync_copy(wv, deg_sh.at[didx], add=True)
            return 0

        lax.fori_loop(0, NTCH, chunk, 0)
        plsc.subcore_barrier()
        off = cid * (NPAD // NC) + sid * OUT_SL
        pltpu.sync_copy(deg_sh.at[pl.ds(off, OUT_SL)],
                        initb.at[pl.ds(0, OUT_SL)])
        pltpu.sync_copy(initb.at[pl.ds(0, OUT_SL)],
                        deg_out.at[pl.ds(off, OUT_SL)])

    return deg_kernel


def _make_agg_kernel(N, D, E_pad, NPAD):
    EPT = E_pad // NW
    B = 128
    NCH = EPT // B
    SUP = 16
    NSUP = NCH // SUP
    SL = NPAD // NS
    mesh = plsc.VectorSubcoreMesh(core_axis_name="c", subcore_axis_name="s")

    @functools.partial(
        pl.kernel,
        out_type=jax.ShapeDtypeStruct((NC, NPAD, D), jnp.float32),
        mesh=mesh,
        scratch_types=[
            pltpu.VMEM_SHARED((NPAD, D), jnp.float32),
            pltpu.VMEM((2, SUP, B), jnp.int32),
            pltpu.VMEM((2, SUP, B), jnp.int32),
            pltpu.VMEM((2, SUP, B), jnp.float32),
            pltpu.VMEM((B, D), jnp.float32),
            pltpu.VMEM((B, D), jnp.float32),
            pltpu.SemaphoreType.DMA,
            pltpu.SemaphoreType.DMA,
            pltpu.SemaphoreType.DMA,
            pltpu.SemaphoreType.DMA,
            pltpu.SemaphoreType.DMA,
            pltpu.SemaphoreType.DMA,
        ],
    )
    def agg_kernel(p_hbm, src_hbm, dst_hbm, w_hbm, q_out,
                   acc_sh, sidx, didx, wv, rows0, rows1,
                   g0, g1, s0, s1, ms0, ms1):
        cid = lax.axis_index("c")
        sid = lax.axis_index("s")
        trow = (cid * NS + sid) * NCH
        msems = (ms0, ms1)

        def stage_meta(sup, par):
            r = pl.multiple_of(trow + sup * SUP, 8)
            sem = msems[par]
            return (
                pltpu.async_copy(src_hbm.at[pl.ds(r, SUP)], sidx.at[par], sem),
                pltpu.async_copy(dst_hbm.at[pl.ds(r, SUP)], didx.at[par], sem),
                pltpu.async_copy(w_hbm.at[pl.ds(r, SUP)], wv.at[par], sem),
            )

        cps = stage_meta(0, 0)

        zero = jnp.zeros((L,), jnp.float32)

        def zrow(r, _):
            for j in range(D // L):
                rows0[r, pl.ds(j * L, L)] = zero
            return 0

        lax.fori_loop(0, B, zrow, 0)
        zcps = [
            pltpu.async_copy(rows0, acc_sh.at[pl.ds(sid * SL + k * B, B)],
                             s0)
            for k in range(SL // B)
        ]
        for cp in zcps:
            cp.wait()
        plsc.subcore_barrier()

        def scale(rbuf, wa, c):
            def sbody(g, _):
                wg = wa[c, pl.ds(g * L, L)]
                for lane in range(L):
                    ws = wg[lane]
                    e = g * L + lane
                    for j in range(D // L):
                        sl = pl.ds(j * L, L)
                        rbuf[e, sl] = rbuf[e, sl] * ws
                return 0

            lax.fori_loop(0, B // L, sbody, 0)

        for sup in range(NSUP):
            par = sup % 2
            sA = sidx.at[par]
            dA = didx.at[par]
            wA = wv.at[par]
            if sup + 1 < NSUP:
                next_cps = stage_meta(sup + 1, 1 - par)
            for cp in cps:
                cp.wait()
            cps = next_cps if sup + 1 < NSUP else ()

            pltpu.async_copy(p_hbm.at[sA.at[0]], rows0, g0)

            def chunk2(cc, _):
                c0 = cc * 2
                c1 = c0 + 1
                pltpu.make_async_copy(p_hbm.at[sA.at[c0]], rows0, g0).wait()

                @pl.when(cc >= 1)
                def _():
                    pltpu.make_async_copy(rows1, acc_sh.at[dA.at[c0 - 1]],
                                          s1).wait()

                pltpu.async_copy(p_hbm.at[sA.at[c1]], rows1, g1)
                scale(rows0, wA, c0)
                pltpu.async_copy(rows0, acc_sh.at[dA.at[c0]], s0, add=True)

                pltpu.make_async_copy(p_hbm.at[sA.at[c1]], rows1, g1).wait()

                @pl.when(cc < SUP // 2 - 1)
                def _():
                    pltpu.make_async_copy(rows0, acc_sh.at[dA.at[c0]],
                                          s0).wait()
                    pltpu.async_copy(p_hbm.at[sA.at[c0 + 2]], rows0, g0)

                scale(rows1, wA, c1)
                pltpu.async_copy(rows1, acc_sh.at[dA.at[c1]], s1, add=True)
                return 0

            lax.fori_loop(0, SUP // 2, chunk2, 0)
            pltpu.make_async_copy(rows0, acc_sh.at[dA.at[SUP - 2]],
                                  s0).wait()
            pltpu.make_async_copy(rows1, acc_sh.at[dA.at[SUP - 1]],
                                  s1).wait()

        plsc.subcore_barrier()
        wbufs = (rows0, rows1)
        wsems = (g0, g1)
        osems = (s0, s1)
        outs = [None, None]
        for k in range(SL // B):
            r0 = sid * SL + k * B
            buf = wbufs[k % 2]
            if outs[k % 2] is not None:
                outs[k % 2].wait()
            pltpu.async_copy(acc_sh.at[pl.ds(r0, B)], buf,
                             wsems[k % 2]).wait()
            outs[k % 2] = pltpu.async_copy(buf, q_out.at[cid, pl.ds(r0, B)],
                                           osems[k % 2])
        for cp in outs:
            cp.wait()

    return agg_kernel


def _tc1(x, W0, degr):
    N, D = x.shape
    BR = 1000
    grid = (N // BR,)

    def body(x_ref, w_ref, deg_ref, p_ref, dis_ref):
        dis = lax.rsqrt(deg_ref[...])
        h = jnp.dot(x_ref[...], w_ref[...], preferred_element_type=jnp.float32)
        p_ref[...] = h * dis
        dis_ref[...] = dis

    return pl.pallas_call(
        body,
        grid=grid,
        in_specs=[
            pl.BlockSpec((BR, D), lambda i: (i, 0)),
            pl.BlockSpec((D, D), lambda i: (0, 0)),
            pl.BlockSpec((BR, 1), lambda i: (i, 0)),
        ],
        out_specs=[
            pl.BlockSpec((BR, D), lambda i: (i, 0)),
            pl.BlockSpec((BR, 1), lambda i: (i, 0)),
        ],
        out_shape=[
            jax.ShapeDtypeStruct((N, D), jnp.float32),
            jax.ShapeDtypeStruct((N, 1), jnp.float32),
        ],
    )(x, W0, degr)


def _tc2(q1, p0, dis, b0, W1):
    N, D = p0.shape
    BR = 1000
    grid = (N // BR,)

    def body(qa_ref, qb_ref, p_ref, dis_ref, b_ref, w_ref, p1_ref):
        dis = dis_ref[...]
        z = dis * (qa_ref[0] + qb_ref[0] + p_ref[...]) + b_ref[...]
        z = jnp.maximum(z, 0.0)
        p1_ref[...] = dis * jnp.dot(z, w_ref[...],
                                    preferred_element_type=jnp.float32)

    return pl.pallas_call(
        body,
        grid=grid,
        in_specs=[
            pl.BlockSpec((1, BR, D), lambda i: (0, i, 0)),
            pl.BlockSpec((1, BR, D), lambda i: (1, i, 0)),
            pl.BlockSpec((BR, D), lambda i: (i, 0)),
            pl.BlockSpec((BR, 1), lambda i: (i, 0)),
            pl.BlockSpec((1, D), lambda i: (0, 0)),
            pl.BlockSpec((D, D), lambda i: (0, 0)),
        ],
        out_specs=pl.BlockSpec((BR, D), lambda i: (i, 0)),
        out_shape=jax.ShapeDtypeStruct((N, D), jnp.float32),
    )(q1, q1, p0, dis, b0, W1)


def _tc3(q2, p1, dis, b1):
    N, D = p1.shape
    BR = 1000
    grid = (N // BR,)

    def body(qa_ref, qb_ref, p_ref, dis_ref, b_ref, o_ref):
        o_ref[...] = (dis_ref[...] * (qa_ref[0] + qb_ref[0] + p_ref[...])
                      + b_ref[...])

    return pl.pallas_call(
        body,
        grid=grid,
        in_specs=[
            pl.BlockSpec((1, BR, D), lambda i: (0, i, 0)),
            pl.BlockSpec((1, BR, D), lambda i: (1, i, 0)),
            pl.BlockSpec((BR, D), lambda i: (i, 0)),
            pl.BlockSpec((BR, 1), lambda i: (i, 0)),
            pl.BlockSpec((1, D), lambda i: (0, 0)),
        ],
        out_specs=pl.BlockSpec((BR, D), lambda i: (i, 0)),
        out_shape=jax.ShapeDtypeStruct((N, D), jnp.float32),
    )(q2, q2, p1, dis, b1)


def kernel(x, edge_index, edge_weight, W0, b0, W1, b1):
    N, D = x.shape
    E = edge_index.shape[1]
    NPAD = _pad_up(N, 1024)
    E_pad = _pad_up(E, NS * 2048)
    pad = E_pad - E

    src = edge_index[0]
    dst = edge_index[1]
    apad = jnp.arange(pad, dtype=dst.dtype)
    srcp = jnp.concatenate([src, apad % N])
    dstp = jnp.concatenate([dst, N + apad % (NPAD - N)])
    wp = jnp.concatenate([edge_weight, jnp.zeros((pad,), edge_weight.dtype)])

    deg = _make_deg_kernel(E_pad, NPAD)(dstp, wp)
    degr = deg[:N].reshape(N, 1)

    p0, dis = _tc1(x, W0, degr)

    agg = _make_agg_kernel(N, D, E_pad, NPAD)
    src2 = srcp.reshape(-1, 128)
    dst2 = dstp.reshape(-1, 128)
    w2 = wp.reshape(-1, 128)
    q1 = agg(p0, src2, dst2, w2)
    p1 = _tc2(q1, p0, dis, b0.reshape(1, D), W1)
    q2 = agg(p1, src2, dst2, w2)
    out = _tc3(q2, p1, dis, b1.reshape(1, D))
    return out


# speedup vs baseline: 27.2562x; 1.0249x over previous
import functools

import jax
import jax.numpy as jnp
from jax import lax
from jax.experimental import pallas as pl
from jax.experimental.pallas import tpu as pltpu
from jax.experimental.pallas import tpu_sc as plsc

NC = 2
NS = 16
L = 16
NW = NC * NS


def _pad_up(v, m):
    return (v + m - 1) // m * m


def _make_deg_kernel(E_pad, NPAD):
    EPT = E_pad // NW
    CH = 2048
    NTCH = EPT // CH
    SL = NPAD // NS
    mesh = plsc.VectorSubcoreMesh(core_axis_name="c", subcore_axis_name="s")

    @functools.partial(
        pl.kernel,
        out_type=jax.ShapeDtypeStruct((NC, NPAD), jnp.float32),
        mesh=mesh,
        scratch_types=[
            pltpu.VMEM_SHARED((NPAD,), jnp.float32),
            pltpu.VMEM((CH,), jnp.int32),
            pltpu.VMEM((CH,), jnp.int32),
            pltpu.VMEM((CH,), jnp.float32),
            pltpu.VMEM((CH,), jnp.float32),
            pltpu.VMEM((SL,), jnp.float32),
            pltpu.SemaphoreType.DMA,
            pltpu.SemaphoreType.DMA,
            pltpu.SemaphoreType.DMA,
        ],
    )
    def deg_kernel(dst_hbm, w_hbm, deg_out, deg_sh, didx0, didx1,
                   wv0, wv1, initb, m0, m1, ssem):
        cid = lax.axis_index("c")
        sid = lax.axis_index("s")
        base = (cid * NS + sid) * EPT
        msems = (m0, m1)
        didxs = (didx0, didx1)
        wvs = (wv0, wv1)

        def stage(ci, par):
            st = pl.multiple_of(base + ci * CH, 8)
            return (
                pltpu.async_copy(dst_hbm.at[pl.ds(st, CH)], didxs[par],
                                 msems[par]),
                pltpu.async_copy(w_hbm.at[pl.ds(st, CH)], wvs[par],
                                 msems[par]),
            )

        cps = stage(0, 0)
        one = jnp.ones((L,), jnp.float32)

        def initbody(i, _):
            initb[pl.ds(i * L, L)] = one * 0.5
            return 0

        lax.fori_loop(0, SL // L, initbody, 0)
        pltpu.sync_copy(initb, deg_sh.at[pl.ds(sid * SL, SL)])
        plsc.subcore_barrier()

        for ci in range(NTCH):
            par = ci % 2
            if ci + 1 < NTCH:
                ncps = stage(ci + 1, 1 - par)
            for cp in cps:
                cp.wait()
            cps = ncps if ci + 1 < NTCH else ()
            pltpu.async_copy(wvs[par], deg_sh.at[didxs[par]], ssem,
                             add=True).wait()
        plsc.subcore_barrier()
        off = sid * SL
        pltpu.sync_copy(deg_sh.at[pl.ds(off, SL)], initb)
        pltpu.sync_copy(initb, deg_out.at[cid, pl.ds(off, SL)])

    return deg_kernel


def _make_agg_kernel(N, D, E_pad, NPAD):
    EPT = E_pad // NW
    B = 128
    NCH = EPT // B
    SUP = 16
    NSUP = NCH // SUP
    SL = NPAD // NS
    mesh = plsc.VectorSubcoreMesh(core_axis_name="c", subcore_axis_name="s")

    @functools.partial(
        pl.kernel,
        out_type=jax.ShapeDtypeStruct((NC, NPAD, D), jnp.float32),
        mesh=mesh,
        scratch_types=[
            pltpu.VMEM_SHARED((NPAD, D), jnp.float32),
            pltpu.VMEM((2, SUP, B), jnp.int32),
            pltpu.VMEM((2, SUP, B), jnp.int32),
            pltpu.VMEM((2, SUP, B), jnp.float32),
            pltpu.VMEM((B, D), jnp.float32),
            pltpu.VMEM((B, D), jnp.float32),
            pltpu.SemaphoreType.DMA,
            pltpu.SemaphoreType.DMA,
            pltpu.SemaphoreType.DMA,
            pltpu.SemaphoreType.DMA,
            pltpu.SemaphoreType.DMA,
            pltpu.SemaphoreType.DMA,
        ],
    )
    def agg_kernel(p_hbm, src_hbm, dst_hbm, w_hbm, q_out,
                   acc_sh, sidx, didx, wv, rows0, rows1,
                   g0, g1, s0, s1, ms0, ms1):
        cid = lax.axis_index("c")
        sid = lax.axis_index("s")
        trow = (cid * NS + sid) * NCH
        msems = (ms0, ms1)

        def stage_meta(sup, par):
            r = pl.multiple_of(trow + sup * SUP, 8)
            sem = msems[par]
            return (
                pltpu.async_copy(src_hbm.at[pl.ds(r, SUP)], sidx.at[par], sem),
                pltpu.async_copy(dst_hbm.at[pl.ds(r, SUP)], didx.at[par], sem),
                pltpu.async_copy(w_hbm.at[pl.ds(r, SUP)], wv.at[par], sem),
            )

        cps = stage_meta(0, 0)

        zero = jnp.zeros((L,), jnp.float32)

        def zrow(r, _):
            for j in range(D // L):
                rows0[r, pl.ds(j * L, L)] = zero
            return 0

        lax.fori_loop(0, B, zrow, 0)
        zcps = [
            pltpu.async_copy(rows0, acc_sh.at[pl.ds(sid * SL + k * B, B)],
                             s0)
            for k in range(SL // B)
        ]
        for cp in zcps:
            cp.wait()
        plsc.subcore_barrier()

        def scale(rbuf, wa, c):
            def sbody(g, _):
                wg = wa[c, pl.ds(g * L, L)]
                for lane in range(L):
                    ws = wg[lane]
                    e = g * L + lane
                    for j in range(D // L):
                        sl = pl.ds(j * L, L)
                        rbuf[e, sl] = rbuf[e, sl] * ws
                return 0

            lax.fori_loop(0, B // L, sbody, 0)

        for sup in range(NSUP):
            par = sup % 2
            sA = sidx.at[par]
            dA = didx.at[par]
            wA = wv.at[par]
            if sup + 1 < NSUP:
                next_cps = stage_meta(sup + 1, 1 - par)
            for cp in cps:
                cp.wait()
            cps = next_cps if sup + 1 < NSUP else ()

            pltpu.async_copy(p_hbm.at[sA.at[0]], rows0, g0)

            def chunk2(cc, _):
                c0 = cc * 2
                c1 = c0 + 1
                pltpu.make_async_copy(p_hbm.at[sA.at[c0]], rows0, g0).wait()

                @pl.when(cc >= 1)
                def _():
                    pltpu.make_async_copy(rows1, acc_sh.at[dA.at[c0 - 1]],
                                          s1).wait()

                pltpu.async_copy(p_hbm.at[sA.at[c1]], rows1, g1)
                scale(rows0, wA, c0)
                pltpu.async_copy(rows0, acc_sh.at[dA.at[c0]], s0, add=True)

                pltpu.make_async_copy(p_hbm.at[sA.at[c1]], rows1, g1).wait()

                @pl.when(cc < SUP // 2 - 1)
                def _():
                    pltpu.make_async_copy(rows0, acc_sh.at[dA.at[c0]],
                                          s0).wait()
                    pltpu.async_copy(p_hbm.at[sA.at[c0 + 2]], rows0, g0)

                scale(rows1, wA, c1)
                pltpu.async_copy(rows1, acc_sh.at[dA.at[c1]], s1, add=True)
                return 0

            lax.fori_loop(0, SUP // 2, chunk2, 0)
            pltpu.make_async_copy(rows0, acc_sh.at[dA.at[SUP - 2]],
                                  s0).wait()
            pltpu.make_async_copy(rows1, acc_sh.at[dA.at[SUP - 1]],
                                  s1).wait()

        plsc.subcore_barrier()
        wbufs = (rows0, rows1)
        wsems = (g0, g1)
        osems = (s0, s1)
        outs = [None, None]
        for k in range(SL // B):
            r0 = sid * SL + k * B
            buf = wbufs[k % 2]
            if outs[k % 2] is not None:
                outs[k % 2].wait()
            pltpu.async_copy(acc_sh.at[pl.ds(r0, B)], buf,
                             wsems[k % 2]).wait()
            outs[k % 2] = pltpu.async_copy(buf, q_out.at[cid, pl.ds(r0, B)],
                                           osems[k % 2])
        for cp in outs:
            cp.wait()

    return agg_kernel


def _tc1a(x, W0):
    N, D = x.shape
    BR = 1000
    grid = (N // BR,)

    def body(x_ref, w_ref, h_ref):
        h_ref[...] = jnp.dot(x_ref[...], w_ref[...],
                             preferred_element_type=jnp.float32)

    return pl.pallas_call(
        body,
        grid=grid,
        in_specs=[
            pl.BlockSpec((BR, D), lambda i: (i, 0)),
            pl.BlockSpec((D, D), lambda i: (0, 0)),
        ],
        out_specs=pl.BlockSpec((BR, D), lambda i: (i, 0)),
        out_shape=jax.ShapeDtypeStruct((N, D), jnp.float32),
    )(x, W0)


def _tc1b(h0, degp):
    N, D = h0.shape
    BR = 1000
    grid = (N // BR,)

    def body(h_ref, d0_ref, d1_ref, p_ref, dis_ref):
        dis = lax.rsqrt(d0_ref[0] + d1_ref[0])
        p_ref[...] = h_ref[...] * dis
        dis_ref[...] = dis

    return pl.pallas_call(
        body,
        grid=grid,
        in_specs=[
            pl.BlockSpec((BR, D), lambda i: (i, 0)),
            pl.BlockSpec((1, BR, 1), lambda i: (0, i, 0)),
            pl.BlockSpec((1, BR, 1), lambda i: (1, i, 0)),
        ],
        out_specs=[
            pl.BlockSpec((BR, D), lambda i: (i, 0)),
            pl.BlockSpec((BR, 1), lambda i: (i, 0)),
        ],
        out_shape=[
            jax.ShapeDtypeStruct((N, D), jnp.float32),
            jax.ShapeDtypeStruct((N, 1), jnp.float32),
        ],
    )(h0, degp, degp)


def _tc2(q1, p0, dis, b0, W1):
    N, D = p0.shape
    BR = 1000
    grid = (N // BR,)

    def body(qa_ref, qb_ref, p_ref, dis_ref, b_ref, w_ref, p1_ref):
        dis = dis_ref[...]
        z = dis * (qa_ref[0] + qb_ref[0] + p_ref[...]) + b_ref[...]
        z = jnp.maximum(z, 0.0)
        p1_ref[...] = dis * jnp.dot(z, w_ref[...],
                                    preferred_element_type=jnp.float32)

    return pl.pallas_call(
        body,
        grid=grid,
        in_specs=[
            pl.BlockSpec((1, BR, D), lambda i: (0, i, 0)),
            pl.BlockSpec((1, BR, D), lambda i: (1, i, 0)),
            pl.BlockSpec((BR, D), lambda i: (i, 0)),
            pl.BlockSpec((BR, 1), lambda i: (i, 0)),
            pl.BlockSpec((1, D), lambda i: (0, 0)),
            pl.BlockSpec((D, D), lambda i: (0, 0)),
        ],
        out_specs=pl.BlockSpec((BR, D), lambda i: (i, 0)),
        out_shape=jax.ShapeDtypeStruct((N, D), jnp.float32),
    )(q1, q1, p0, dis, b0, W1)


def _tc3(q2, p1, dis, b1):
    N, D = p1.shape
    BR = 1000
    grid = (N // BR,)

    def body(qa_ref, qb_ref, p_ref, dis_ref, b_ref, o_ref):
        o_ref[...] = (dis_ref[...] * (qa_ref[0] + qb_ref[0] + p_ref[...])
                      + b_ref[...])

    return pl.pallas_call(
        body,
        grid=grid,
        in_specs=[
            pl.BlockSpec((1, BR, D), lambda i: (0, i, 0)),
            pl.BlockSpec((1, BR, D), lambda i: (1, i, 0)),
            pl.BlockSpec((BR, D), lambda i: (i, 0)),
            pl.BlockSpec((BR, 1), lambda i: (i, 0)),
            pl.BlockSpec((1, D), lambda i: (0, 0)),
        ],
        out_specs=pl.BlockSpec((BR, D), lambda i: (i, 0)),
        out_shape=jax.ShapeDtypeStruct((N, D), jnp.float32),
    )(q2, q2, p1, dis, b1)


def kernel(x, edge_index, edge_weight, W0, b0, W1, b1):
    N, D = x.shape
    E = edge_index.shape[1]
    NPAD = _pad_up(N, 1024)
    E_pad = _pad_up(E, NS * 2048)
    pad = E_pad - E

    src = edge_index[0]
    dst = edge_index[1]
    apad = jnp.arange(pad, dtype=dst.dtype)
    srcp = jnp.concatenate([src, apad % N])
    dstp = jnp.concatenate([dst, N + apad % (NPAD - N)])
    wp = jnp.concatenate([edge_weight, jnp.zeros((pad,), edge_weight.dtype)])

    deg = _make_deg_kernel(E_pad, NPAD)(dstp, wp)
    degp = deg[:, :N].reshape(2, N, 1)

    h0 = _tc1a(x, W0)
    p0, dis = _tc1b(h0, degp)

    agg = _make_agg_kernel(N, D, E_pad, NPAD)
    src2 = srcp.reshape(-1, 128)
    dst2 = dstp.reshape(-1, 128)
    w2 = wp.reshape(-1, 128)
    q1 = agg(p0, src2, dst2, w2)
    p1 = _tc2(q1, p0, dis, b0.reshape(1, D), W1)
    q2 = agg(p1, src2, dst2, w2)
    out = _tc3(q2, p1, dis, b1.reshape(1, D))
    return out
